# Initial kernel scaffold; baseline (speedup 1.0000x reference)
import functools

import jax
import jax.numpy as jnp
from jax import lax
from jax.experimental import pallas as pl
from jax.experimental.pallas import tpu as pltpu
from jax.experimental.pallas import tpu_sc as plsc

N_NODES = 10000
N_EDGES = 160000
D_FEAT = 256
D_HID = 32
DH2 = 2 * D_HID

NC = 2
NS = 16
NW = NC * NS
EDGES_PER_TILE = N_EDGES // NW
CHUNK = 1000
NCHUNK = EDGES_PER_TILE // CHUNK
NPAD = 10240
ROWS_PER_TILE = NPAD // NS

_SC_MESH = dict(core_axis_name="c", subcore_axis_name="s",
                num_cores=NC, num_subcores=NS)


def _deg_body(dst_hbm, zeros_hbm, ones_hbm, out_hbm, idx_v, ones_v, acc_sh):
    cid = lax.axis_index("c")
    sid = lax.axis_index("s")
    row0 = sid * ROWS_PER_TILE
    pltpu.sync_copy(zeros_hbm.at[cid, sid], acc_sh.at[pl.ds(row0, ROWS_PER_TILE)])
    pltpu.sync_copy(ones_hbm.at[cid, sid], ones_v)
    plsc.subcore_barrier()
    for j in range(NCHUNK):
        pltpu.sync_copy(dst_hbm.at[cid, sid, j], idx_v)
        pltpu.sync_copy(ones_v, acc_sh.at[idx_v], add=True)
    plsc.subcore_barrier()
    pltpu.sync_copy(acc_sh.at[pl.ds(row0, ROWS_PER_TILE)],
                    out_hbm.at[cid, pl.ds(row0, ROWS_PER_TILE)])


_deg_kernel = functools.partial(
    pl.kernel,
    out_type=jax.ShapeDtypeStruct((NC, NPAD), jnp.float32),
    mesh=plsc.VectorSubcoreMesh(**_SC_MESH),
    scratch_types=[
        pltpu.VMEM((CHUNK,), jnp.int32),
        pltpu.VMEM((CHUNK,), jnp.float32),
        pltpu.VMEM_SHARED((NPAD,), jnp.float32),
    ],
)(_deg_body)


def _agg_body(y_hbm, src_hbm, dst_hbm, out_hbm, src_v, dst_v, rows_v, acc_sh, sem):
    cid = lax.axis_index("c")
    sid = lax.axis_index("s")
    row0 = sid * ROWS_PER_TILE
    pltpu.sync_copy(y_hbm.at[pl.ds(row0, ROWS_PER_TILE)],
                    acc_sh.at[pl.ds(row0, ROWS_PER_TILE)])
    plsc.subcore_barrier()
    for j in range(NCHUNK):
        pltpu.sync_copy(src_hbm.at[cid, sid, j], src_v)
        pltpu.sync_copy(dst_hbm.at[cid, sid, j], dst_v)
        pltpu.async_copy(y_hbm.at[src_v], rows_v, sem).wait()
        pltpu.sync_copy(rows_v, acc_sh.at[dst_v], add=True)
    plsc.subcore_barrier()
    pltpu.sync_copy(acc_sh.at[pl.ds(row0, ROWS_PER_TILE)],
                    out_hbm.at[cid, pl.ds(row0, ROWS_PER_TILE)])


_agg_kernel = functools.partial(
    pl.kernel,
    out_type=jax.ShapeDtypeStruct((NC, NPAD, DH2), jnp.float32),
    mesh=plsc.VectorSubcoreMesh(**_SC_MESH),
    scratch_types=[
        pltpu.VMEM((CHUNK,), jnp.int32),
        pltpu.VMEM((CHUNK,), jnp.int32),
        pltpu.VMEM((CHUNK, DH2), jnp.float32),
        pltpu.VMEM_SHARED((NPAD, DH2), jnp.float32),
        pltpu.SemaphoreType.DMA,
    ],
)(_agg_body)


_ROWS_BLK = 1024
_N_BLKS = NPAD // _ROWS_BLK


def _proj_body(x_ref, w_ref, degp_ref, y_ref):
    deg = degp_ref[0, :] + degp_ref[1, :] + 1.0
    dis = lax.rsqrt(deg)
    xw = jnp.dot(x_ref[...], w_ref[...], preferred_element_type=jnp.float32)
    y_ref[...] = xw * dis[:, None]


def _proj(x2p, wcat, degp):
    return pl.pallas_call(
        _proj_body,
        grid=(_N_BLKS,),
        in_specs=[
            pl.BlockSpec((_ROWS_BLK, D_FEAT), lambda i: (i, 0)),
            pl.BlockSpec((D_FEAT, DH2), lambda i: (0, 0)),
            pl.BlockSpec((NC, _ROWS_BLK), lambda i: (0, i)),
        ],
        out_specs=pl.BlockSpec((_ROWS_BLK, DH2), lambda i: (i, 0)),
        out_shape=jax.ShapeDtypeStruct((NPAD, DH2), jnp.float32),
    )(x2p, wcat, degp)


def _fin_body(acc_ref, y_ref, degp_ref, m1_ref, m2_ref, bz_ref, bh_ref,
              wl_ref, bl_ref, out_ref):
    deg = degp_ref[0, :] + degp_ref[1, :] + 1.0
    dis = lax.rsqrt(deg)
    t = (acc_ref[0] + acc_ref[1] - y_ref[...]) * dis[:, None]
    zin = jnp.dot(t, m1_ref[...], preferred_element_type=jnp.float32) + bz_ref[...]
    hin = jnp.dot(t, m2_ref[...], preferred_element_type=jnp.float32) + bh_ref[...]
    z = jax.nn.sigmoid(zin)
    ht = jnp.tanh(hin)
    f = (1.0 - z) * ht
    out_ref[...] = jnp.sum(f * wl_ref[...], axis=1) + bl_ref[0, 0]


def _final(acc, y, degp, m1, m2, bz, bh, wlrow, blin):
    return pl.pallas_call(
        _fin_body,
        grid=(_N_BLKS,),
        in_specs=[
            pl.BlockSpec((NC, _ROWS_BLK, DH2), lambda i: (0, i, 0)),
            pl.BlockSpec((_ROWS_BLK, DH2), lambda i: (i, 0)),
            pl.BlockSpec((NC, _ROWS_BLK), lambda i: (0, i)),
            pl.BlockSpec((DH2, D_HID), lambda i: (0, 0)),
            pl.BlockSpec((DH2, D_HID), lambda i: (0, 0)),
            pl.BlockSpec((1, D_HID), lambda i: (0, 0)),
            pl.BlockSpec((1, D_HID), lambda i: (0, 0)),
            pl.BlockSpec((1, D_HID), lambda i: (0, 0)),
            pl.BlockSpec((1, 1), lambda i: (0, 0)),
        ],
        out_specs=pl.BlockSpec((_ROWS_BLK,), lambda i: (i,)),
        out_shape=jax.ShapeDtypeStruct((NPAD,), jnp.float32),
    )(acc, y, degp, m1, m2, bz, bh, wlrow, blin)


def kernel(x, edge_index, W_z, b_z, W_r, b_r, W_h, b_h, lz_W, lz_b,
           lr_W, lr_b, lh_W, lh_b, att, W_lin, b_lin):
    f32 = jnp.float32
    x2 = x[:, :, 0]
    x2p = jnp.pad(x2, ((0, NPAD - N_NODES), (0, 0)))
    wcat = jnp.concatenate([W_z, W_h], axis=1)

    src3 = edge_index[0].reshape(NC, NS, NCHUNK, CHUNK)
    dst3 = edge_index[1].reshape(NC, NS, NCHUNK, CHUNK)

    zeros_init = jnp.zeros((NC, NS, ROWS_PER_TILE), f32)
    ones_vals = jnp.ones((NC, NS, CHUNK), f32)

    degp = _deg_kernel(dst3, zeros_init, ones_vals)
    y = _proj(x2p, wcat, degp)
    acc = _agg_kernel(y, src3, dst3)

    zeros32 = jnp.zeros((D_HID, D_HID), f32)
    m1 = jnp.concatenate([lz_W[:D_HID], zeros32], axis=0)
    m2 = jnp.concatenate([zeros32, lh_W[:D_HID]], axis=0)
    bz = (b_z @ lz_W[:D_HID] + lz_b).reshape(1, D_HID)
    bh = (b_h @ lh_W[:D_HID] + lh_b).reshape(1, D_HID)
    wlrow = W_lin[:, 0].reshape(1, D_HID)
    blin = b_lin.reshape(1, 1)

    out = _final(acc, y, degp, m1, m2, bz, bh, wlrow, blin)
    return out[:N_NODES]


# speedup vs baseline: 43.6552x; 43.6552x over previous
import functools

import jax
import jax.numpy as jnp
from jax import lax
from jax.experimental import pallas as pl
from jax.experimental.pallas import tpu as pltpu
from jax.experimental.pallas import tpu_sc as plsc

N_NODES = 10000
N_EDGES = 160000
D_FEAT = 256
D_HID = 32
DH2 = 2 * D_HID

NC = 2
NS = 16
NW = NC * NS
EDGES_PER_TILE = N_EDGES // NW
CHUNK = 1000
NCHUNK = EDGES_PER_TILE // CHUNK
NPAD = 10240
ROWS_PER_TILE = NPAD // NS

_SC_MESH = dict(core_axis_name="c", subcore_axis_name="s",
                num_cores=NC, num_subcores=NS)


def _deg_body(dst_hbm, zeros_hbm, ones_hbm, out_hbm, idx_v, ones_v, acc_sh):
    cid = lax.axis_index("c")
    sid = lax.axis_index("s")
    wid = cid * NS + sid
    row0 = sid * ROWS_PER_TILE
    pltpu.sync_copy(zeros_hbm.at[pl.ds(wid * ROWS_PER_TILE, ROWS_PER_TILE)],
                    acc_sh.at[pl.ds(row0, ROWS_PER_TILE)])
    pltpu.sync_copy(ones_hbm.at[pl.ds(wid * CHUNK, CHUNK)], ones_v)
    plsc.subcore_barrier()
    for j in range(NCHUNK):
        base = (wid * NCHUNK + j) * CHUNK
        pltpu.sync_copy(dst_hbm.at[pl.ds(base, CHUNK)], idx_v)
        pltpu.sync_copy(ones_v, acc_sh.at[idx_v], add=True)
    plsc.subcore_barrier()
    pltpu.sync_copy(acc_sh.at[pl.ds(row0, ROWS_PER_TILE)],
                    out_hbm.at[pl.ds(cid * NPAD + row0, ROWS_PER_TILE)])


_deg_kernel = functools.partial(
    pl.kernel,
    out_type=jax.ShapeDtypeStruct((NC * NPAD,), jnp.float32),
    mesh=plsc.VectorSubcoreMesh(**_SC_MESH),
    scratch_types=[
        pltpu.VMEM((CHUNK,), jnp.int32),
        pltpu.VMEM((CHUNK,), jnp.float32),
        pltpu.VMEM_SHARED((NPAD,), jnp.float32),
    ],
)(_deg_body)


def _agg_body(y_hbm, src_hbm, dst_hbm, out_hbm, src_v, dst_v, rows_v, acc_sh, sem):
    cid = lax.axis_index("c")
    sid = lax.axis_index("s")
    row0 = sid * ROWS_PER_TILE
    pltpu.sync_copy(y_hbm.at[pl.ds(row0, ROWS_PER_TILE)],
                    acc_sh.at[pl.ds(row0, ROWS_PER_TILE)])
    plsc.subcore_barrier()
    wid = cid * NS + sid
    for j in range(NCHUNK):
        base = (wid * NCHUNK + j) * CHUNK
        pltpu.sync_copy(src_hbm.at[pl.ds(base, CHUNK)], src_v)
        pltpu.sync_copy(dst_hbm.at[pl.ds(base, CHUNK)], dst_v)
        pltpu.async_copy(y_hbm.at[src_v], rows_v, sem).wait()
        pltpu.sync_copy(rows_v, acc_sh.at[dst_v], add=True)
    plsc.subcore_barrier()
    pltpu.sync_copy(acc_sh.at[pl.ds(row0, ROWS_PER_TILE)],
                    out_hbm.at[cid, pl.ds(row0, ROWS_PER_TILE)])


_agg_kernel = functools.partial(
    pl.kernel,
    out_type=jax.ShapeDtypeStruct((NC, NPAD, DH2), jnp.float32),
    mesh=plsc.VectorSubcoreMesh(**_SC_MESH),
    compiler_params=pltpu.CompilerParams(use_tc_tiling_on_sc=False),
    scratch_types=[
        pltpu.VMEM((CHUNK,), jnp.int32),
        pltpu.VMEM((CHUNK,), jnp.int32),
        pltpu.VMEM((CHUNK, DH2), jnp.float32),
        pltpu.VMEM_SHARED((NPAD, DH2), jnp.float32),
        pltpu.SemaphoreType.DMA,
    ],
)(_agg_body)


_ROWS_BLK = 1024
_N_BLKS = NPAD // _ROWS_BLK


def _proj_body(x_ref, w_ref, degp_ref, y_ref):
    deg = degp_ref[0, :] + degp_ref[1, :] + 1.0
    dis = lax.rsqrt(deg)
    xw = jnp.dot(x_ref[...], w_ref[...], preferred_element_type=jnp.float32)
    y_ref[...] = xw * dis[:, None]


def _proj(x2p, wcat, degp):
    return pl.pallas_call(
        _proj_body,
        grid=(_N_BLKS,),
        in_specs=[
            pl.BlockSpec((_ROWS_BLK, D_FEAT), lambda i: (i, 0)),
            pl.BlockSpec((D_FEAT, DH2), lambda i: (0, 0)),
            pl.BlockSpec((NC, _ROWS_BLK), lambda i: (0, i)),
        ],
        out_specs=pl.BlockSpec((_ROWS_BLK, DH2), lambda i: (i, 0)),
        out_shape=jax.ShapeDtypeStruct((NPAD, DH2), jnp.float32),
    )(x2p, wcat, degp)


def _fin_body(acc_ref, y_ref, degp_ref, m1_ref, m2_ref, bz_ref, bh_ref,
              wl_ref, bl_ref, out_ref):
    deg = degp_ref[0, :] + degp_ref[1, :] + 1.0
    dis = lax.rsqrt(deg)
    t = (acc_ref[0] + acc_ref[1] - y_ref[...]) * dis[:, None]
    zin = jnp.dot(t, m1_ref[...], preferred_element_type=jnp.float32) + bz_ref[...]
    hin = jnp.dot(t, m2_ref[...], preferred_element_type=jnp.float32) + bh_ref[...]
    z = jax.nn.sigmoid(zin)
    ht = jnp.tanh(hin)
    f = (1.0 - z) * ht
    out_ref[...] = jnp.sum(f * wl_ref[...], axis=1) + bl_ref[0, 0]


def _final(acc, y, degp, m1, m2, bz, bh, wlrow, blin):
    return pl.pallas_call(
        _fin_body,
        grid=(_N_BLKS,),
        in_specs=[
            pl.BlockSpec((NC, _ROWS_BLK, DH2), lambda i: (0, i, 0)),
            pl.BlockSpec((_ROWS_BLK, DH2), lambda i: (i, 0)),
            pl.BlockSpec((NC, _ROWS_BLK), lambda i: (0, i)),
            pl.BlockSpec((DH2, D_HID), lambda i: (0, 0)),
            pl.BlockSpec((DH2, D_HID), lambda i: (0, 0)),
            pl.BlockSpec((1, D_HID), lambda i: (0, 0)),
            pl.BlockSpec((1, D_HID), lambda i: (0, 0)),
            pl.BlockSpec((1, D_HID), lambda i: (0, 0)),
            pl.BlockSpec((1, 1), lambda i: (0, 0)),
        ],
        out_specs=pl.BlockSpec((_ROWS_BLK,), lambda i: (i,)),
        out_shape=jax.ShapeDtypeStruct((NPAD,), jnp.float32),
    )(acc, y, degp, m1, m2, bz, bh, wlrow, blin)


def kernel(x, edge_index, W_z, b_z, W_r, b_r, W_h, b_h, lz_W, lz_b,
           lr_W, lr_b, lh_W, lh_b, att, W_lin, b_lin):
    f32 = jnp.float32
    x2 = x[:, :, 0]
    x2p = jnp.pad(x2, ((0, NPAD - N_NODES), (0, 0)))
    wcat = jnp.concatenate([W_z, W_h], axis=1)

    src_flat = edge_index[0]
    dst_flat = edge_index[1]

    zeros_init = jnp.zeros((NW * ROWS_PER_TILE,), f32)
    ones_vals = jnp.ones((NW * CHUNK,), f32)

    degp = _deg_kernel(dst_flat, zeros_init, ones_vals).reshape(NC, NPAD)
    y = _proj(x2p, wcat, degp)
    acc = _agg_kernel(y, src_flat, dst_flat)

    zeros32 = jnp.zeros((D_HID, D_HID), f32)
    m1 = jnp.concatenate([lz_W[:D_HID], zeros32], axis=0)
    m2 = jnp.concatenate([zeros32, lh_W[:D_HID]], axis=0)
    bz = (b_z @ lz_W[:D_HID] + lz_b).reshape(1, D_HID)
    bh = (b_h @ lh_W[:D_HID] + lh_b).reshape(1, D_HID)
    wlrow = W_lin[:, 0].reshape(1, D_HID)
    blin = b_lin.reshape(1, 1)

    out = _final(acc, y, degp, m1, m2, bz, bh, wlrow, blin)
    return out[:N_NODES]
